# baseline (speedup 1.0000x reference)
import functools

import jax
import jax.numpy as jnp
from jax import lax
from jax.experimental import pallas as pl
from jax.experimental.pallas import tpu as pltpu
from jax.experimental.pallas import tpu_sc as plsc

RES = 512
H = 1080
W = 1920
NPX = H * W
NWORKERS = 32
PX_PER_W = NPX // NWORKERS
C = 3600
NCHUNK = PX_PER_W // C
VPC = C // 16
GROUPS = [128] * (C // 128) + ([C % 128] if C % 128 else [])


def _build_window_table(cubemap):
    t = jnp.pad(cubemap, ((0, 0), (0, 1), (0, 1), (0, 1)))
    w00 = t[:, :RES, :RES, :]
    w01 = t[:, :RES, 1:RES + 1, :]
    w10 = t[:, 1:RES + 1, :RES, :]
    w11 = t[:, 1:RES + 1, 1:RES + 1, :]
    win = jnp.stack([w00, w01, w10, w11], axis=3)
    return win.reshape(6 * RES * RES, 16)


def _sc_body(rays_hbm, table_hbm, out_hbm,
             rays_v, idx_v, wx_v, wy_v, win_v, out_v, sem):
    wid = lax.axis_index("s") * 2 + lax.axis_index("c")
    iota = lax.iota(jnp.int32, 16)
    iota3 = iota * 3

    def chunk_body(ci, carry):
        base_px = wid * PX_PER_W + ci * C
        pltpu.sync_copy(rays_hbm.at[pl.ds(base_px * 3, C * 3)], rays_v)

        def vec_body(i, carry2):
            ix = iota3 + i * 48
            xx = plsc.load_gather(rays_v, [ix])
            yy = plsc.load_gather(rays_v, [ix + 1])
            zz = plsc.load_gather(rays_v, [ix + 2])
            ax, ay, az = jnp.abs(xx), jnp.abs(yy), jnp.abs(zz)
            px, py, pz = xx >= 0.0, yy >= 0.0, zz >= 0.0
            is_x = (ax >= ay) & (ax >= az)
            is_y = (~is_x) & (ay >= az)
            face = jnp.where(
                is_x, jnp.where(px, 0, 1),
                jnp.where(is_y, jnp.where(py, 2, 3), jnp.where(pz, 4, 5)))
            ma = jnp.maximum(jnp.where(is_x, ax, jnp.where(is_y, ay, az)),
                             1e-12)
            sc_ = jnp.where(is_x, jnp.where(px, -zz, zz),
                            jnp.where(is_y, xx, jnp.where(pz, xx, -xx)))
            tc_ = jnp.where(is_x, -yy,
                            jnp.where(is_y, jnp.where(py, zz, -zz), -yy))
            inv = 1.0 / ma
            fx = (sc_ * inv + 1.0) * (0.5 * RES) - 0.5
            fy = (tc_ * inv + 1.0) * (0.5 * RES) - 0.5
            xb = jnp.clip(fx.astype(jnp.int32), 0, RES - 2)
            yb = jnp.clip(fy.astype(jnp.int32), 0, RES - 2)
            wx = jnp.clip(fx, 0.0, RES - 1.0) - xb.astype(jnp.float32)
            wy = jnp.clip(fy, 0.0, RES - 1.0) - yb.astype(jnp.float32)
            s = pl.ds(i * 16, 16)
            idx_v[s] = (face << 18) | (yb << 9) | xb
            wx_v[s] = wx
            wy_v[s] = wy
            return carry2

        lax.fori_loop(0, VPC, vec_body, 0, unroll=2)

        handles = []
        off = 0
        for g in GROUPS:
            handles.append(pltpu.async_copy(
                table_hbm.at[idx_v.at[pl.ds(off, g)]],
                win_v.at[pl.ds(off, g)], sem))
            off += g
        for h in handles:
            h.wait()

        def blend_body(i, carry2):
            s = pl.ds(i * 16, 16)
            rows = iota + i * 16
            wx = wx_v[s]
            wy = wy_v[s]
            for ch in range(3):
                c00 = plsc.load_gather(win_v, [rows, iota * 0 + ch])
                c01 = plsc.load_gather(win_v, [rows, iota * 0 + (4 + ch)])
                c10 = plsc.load_gather(win_v, [rows, iota * 0 + (8 + ch)])
                c11 = plsc.load_gather(win_v, [rows, iota * 0 + (12 + ch)])
                top = c00 + wx * (c01 - c00)
                bot = c10 + wx * (c11 - c10)
                o = top + wy * (bot - top)
                out_v[pl.ds(ch * C + i * 16, 16)] = jnp.clip(o, 0.0, 1.0)
            return carry2

        lax.fori_loop(0, VPC, blend_body, 0, unroll=2)

        for ch in range(3):
            pltpu.sync_copy(out_v.at[pl.ds(ch * C, C)],
                            out_hbm.at[pl.ds(ch * NPX + base_px, C)])
        return carry

    lax.fori_loop(0, NCHUNK, chunk_body, 0)


@jax.jit
def kernel(rays_d, sky_cube_map):
    table = _build_window_table(sky_cube_map)
    rays_flat = rays_d.reshape(NPX * 3)

    sc_fn = functools.partial(
        pl.kernel,
        mesh=plsc.VectorSubcoreMesh(core_axis_name="c", subcore_axis_name="s"),
        compiler_params=pltpu.CompilerParams(needs_layout_passes=False,
                                             use_tc_tiling_on_sc=False),
        out_type=jax.ShapeDtypeStruct((3 * NPX,), jnp.float32),
        scratch_types=[
            pltpu.VMEM((C * 3,), jnp.float32),
            pltpu.VMEM((C,), jnp.int32),
            pltpu.VMEM((C,), jnp.float32),
            pltpu.VMEM((C,), jnp.float32),
            pltpu.VMEM((C, 16), jnp.float32),
            pltpu.VMEM((3 * C,), jnp.float32),
            pltpu.SemaphoreType.DMA,
        ],
    )(_sc_body)
    out = sc_fn(rays_flat, table)
    return out.reshape(3, H, W)
